# speedup vs baseline: 3.2231x; 3.2231x over previous
import functools

import jax
import jax.numpy as jnp
from jax import lax
from jax.experimental import pallas as pl
from jax.experimental.pallas import tpu as pltpu
from jax.experimental.pallas import tpu_sc as plsc

B, L, D = 16384, 50, 128
VOCAB = 24
MAX_CHARGE = 10
ROWS = L * VOCAB * MAX_CHARGE
CAT = L + VOCAB + MAX_CHARGE

NC, NS = 2, 16
NW = NC * NS
TOK = B * L
ROWS_PER_W = TOK // NW
PIECE = 128
NPIECE = ROWS_PER_W // PIECE


def _pos_enc():
    pos = jnp.arange(L, dtype=jnp.float32)[:, None]
    i = jnp.arange(D // 2, dtype=jnp.float32)[None, :]
    angle = pos / jnp.power(10000.0, (2.0 * i) / D)
    return jnp.stack([jnp.sin(angle), jnp.cos(angle)], axis=-1).reshape(L, D)


def _tc_prep(cat_ref, tok_ref, ch_ref, mega_ref, idx_ref):
    r = lax.broadcasted_iota(jnp.int32, (ROWS, CAT), 0)
    col = lax.broadcasted_iota(jnp.int32, (ROWS, CAT), 1)
    l = r // (VOCAB * MAX_CHARGE)
    v = (r // MAX_CHARGE) % VOCAB
    c = r % MAX_CHARGE
    oh = ((col == l) | (col == L + v) | (col == L + VOCAB + c)).astype(jnp.float32)
    mega_ref[...] = jnp.dot(oh, cat_ref[...], preferred_element_type=jnp.float32)
    li = lax.broadcasted_iota(jnp.int32, (B, L), 1)
    idx_ref[...] = li * (VOCAB * MAX_CHARGE) + tok_ref[...] * MAX_CHARGE + ch_ref[...]


def _sc_gather(mega_hbm, idx_hbm, out_hbm, idx_v, buf0, buf1, g0, g1, s0, s1):
    wid = lax.axis_index("s") * NC + lax.axis_index("c")
    base = wid * ROWS_PER_W
    pltpu.sync_copy(idx_hbm.at[pl.ds(wid * NPIECE, NPIECE)], idx_v)

    def body(i, _):
        p0 = 2 * i
        p1 = 2 * i + 1

        @pl.when(i > 0)
        def _():
            pltpu.make_async_copy(buf0, out_hbm.at[pl.ds(base, PIECE)], s0).wait()

        pltpu.async_copy(mega_hbm.at[idx_v.at[p0]], buf0, g0).wait()
        pltpu.async_copy(buf0, out_hbm.at[pl.ds(base + p0 * PIECE, PIECE)], s0)

        @pl.when(i > 0)
        def _():
            pltpu.make_async_copy(buf1, out_hbm.at[pl.ds(base, PIECE)], s1).wait()

        pltpu.async_copy(mega_hbm.at[idx_v.at[p1]], buf1, g1).wait()
        pltpu.async_copy(buf1, out_hbm.at[pl.ds(base + p1 * PIECE, PIECE)], s1)
        return 0

    lax.fori_loop(0, NPIECE // 2, body, 0)
    pltpu.make_async_copy(buf0, out_hbm.at[pl.ds(base, PIECE)], s0).wait()
    pltpu.make_async_copy(buf1, out_hbm.at[pl.ds(base, PIECE)], s1).wait()


def kernel(tokens, charges, aa_table, charge_table):
    cat = jnp.concatenate([_pos_enc(), aa_table, charge_table], axis=0)
    mega, idx = pl.pallas_call(
        _tc_prep,
        out_shape=[
            jax.ShapeDtypeStruct((ROWS, D), jnp.float32),
            jax.ShapeDtypeStruct((B, L), jnp.int32),
        ],
    )(cat, tokens, charges.reshape(B, 1))
    idx2d = idx.reshape(TOK // PIECE, PIECE)

    sc = functools.partial(
        pl.kernel,
        out_type=jax.ShapeDtypeStruct((TOK, D), jnp.float32),
        mesh=plsc.VectorSubcoreMesh(core_axis_name="c", subcore_axis_name="s"),
        scratch_types=[
            pltpu.VMEM((NPIECE, PIECE), jnp.int32),
            pltpu.VMEM((PIECE, D), jnp.float32),
            pltpu.VMEM((PIECE, D), jnp.float32),
            pltpu.SemaphoreType.DMA,
            pltpu.SemaphoreType.DMA,
            pltpu.SemaphoreType.DMA,
            pltpu.SemaphoreType.DMA,
        ],
    )(_sc_gather)
    out = sc(mega, idx2d)
    return out.reshape(B, L, D)


# speedup vs baseline: 5.9328x; 1.8407x over previous
import functools

import jax
import jax.numpy as jnp
from jax import lax
from jax.experimental import pallas as pl
from jax.experimental.pallas import tpu as pltpu
from jax.experimental.pallas import tpu_sc as plsc

B, L, D = 16384, 50, 128
VOCAB = 24
MAX_CHARGE = 10
ROWS = L * VOCAB * MAX_CHARGE
CAT = L + VOCAB + MAX_CHARGE

NC, NS = 2, 16
NW = NC * NS
TOK = B * L
B_PER_W = B // NW
TOK_PER_W = B_PER_W * L
PIECE_B = 4
PTOK = PIECE_B * L
GCHUNKS = ((0, 128), (128, 72))
NPIECE = B_PER_W // PIECE_B


def _pos_enc():
    pos = jnp.arange(L, dtype=jnp.float32)[:, None]
    i = jnp.arange(D // 2, dtype=jnp.float32)[None, :]
    angle = pos / jnp.power(10000.0, (2.0 * i) / D)
    return jnp.stack([jnp.sin(angle), jnp.cos(angle)], axis=-1).reshape(L, D)


def _tc_prep(cat_ref, tok_ref, ch_ref, mega_ref, idx_ref):
    r = lax.broadcasted_iota(jnp.int32, (ROWS, CAT), 0)
    col = lax.broadcasted_iota(jnp.int32, (ROWS, CAT), 1)
    l = r // (VOCAB * MAX_CHARGE)
    v = (r // MAX_CHARGE) % VOCAB
    c = r % MAX_CHARGE
    oh = ((col == l) | (col == L + v) | (col == L + VOCAB + c)).astype(jnp.float32)
    mega_ref[...] = jnp.dot(oh, cat_ref[...], preferred_element_type=jnp.float32)
    li = lax.broadcasted_iota(jnp.int32, (B, L), 1)
    idx_ref[...] = li * (VOCAB * MAX_CHARGE) + tok_ref[...] * MAX_CHARGE + ch_ref[...]


def _sc_gather(mega_hbm, idx_hbm, out_hbm, idx_v, buf0, buf1, g0, g1, s0, s1):
    wid = lax.axis_index("s") * NC + lax.axis_index("c")
    b_base = wid * B_PER_W
    pltpu.sync_copy(idx_hbm.at[pl.ds(wid * TOK_PER_W, TOK_PER_W)], idx_v)

    def piece(i, p, buf, gsem, ssem):
        @pl.when(i > 0)
        def _():
            for _ in range(PIECE_B):
                pltpu.make_async_copy(buf.at[pl.ds(0, L)], out_hbm.at[0], ssem).wait()

        t0 = p * PTOK
        gs = [
            pltpu.async_copy(
                mega_hbm.at[idx_v.at[pl.ds(t0 + off, n)]], buf.at[pl.ds(off, n)], gsem
            )
            for off, n in GCHUNKS
        ]
        for g in gs:
            g.wait()
        b0 = b_base + p * PIECE_B
        for j in range(PIECE_B):
            pltpu.async_copy(buf.at[pl.ds(j * L, L)], out_hbm.at[b0 + j], ssem)

    def body(i, _):
        piece(i, 2 * i, buf0, g0, s0)
        piece(i, 2 * i + 1, buf1, g1, s1)
        return 0

    lax.fori_loop(0, NPIECE // 2, body, 0)
    for buf, ssem in ((buf0, s0), (buf1, s1)):
        for _ in range(PIECE_B):
            pltpu.make_async_copy(buf.at[pl.ds(0, L)], out_hbm.at[0], ssem).wait()


def kernel(tokens, charges, aa_table, charge_table):
    cat = jnp.concatenate([_pos_enc(), aa_table, charge_table], axis=0)
    mega, idx = pl.pallas_call(
        _tc_prep,
        out_shape=[
            jax.ShapeDtypeStruct((ROWS, D), jnp.float32),
            jax.ShapeDtypeStruct((B, L), jnp.int32),
        ],
    )(cat, tokens, charges.reshape(B, 1))
    idx1d = idx.reshape(TOK)

    sc = functools.partial(
        pl.kernel,
        out_type=jax.ShapeDtypeStruct((B, L, D), jnp.float32),
        mesh=plsc.VectorSubcoreMesh(core_axis_name="c", subcore_axis_name="s"),
        scratch_types=[
            pltpu.VMEM((TOK_PER_W,), jnp.int32),
            pltpu.VMEM((PTOK, D), jnp.float32),
            pltpu.VMEM((PTOK, D), jnp.float32),
            pltpu.SemaphoreType.DMA,
            pltpu.SemaphoreType.DMA,
            pltpu.SemaphoreType.DMA,
            pltpu.SemaphoreType.DMA,
        ],
    )(_sc_gather)
    return sc(mega, idx1d)
